# baseline (speedup 1.0000x reference)
import functools

import jax
import jax.numpy as jnp
from jax import lax
from jax.experimental import pallas as pl
from jax.experimental.pallas import tpu as pltpu


def _chunk_t(seq_len, target=16):
    c = min(seq_len, target)
    while seq_len % c:
        c -= 1
    return c


def _fused_encoder_kernel(emb_ref, w_in_ref, b_in_ref, w_ih_ref, b_ref,
                          w_hh_ref, w_out_ref, b_out_ref,
                          out_ref, hid_ref,
                          act_ref, xh_ref, hbuf_ref,
                          *, bh, nhid, chunk_t, n_chunks, nlayers):
    l = pl.program_id(1)
    rows_c = chunk_t * bh

    @pl.when(l == 0)
    def _():
        for c in range(n_chunks):
            sl = pl.ds(c * rows_c, rows_c)
            act_ref[sl, :] = (
                jnp.dot(emb_ref[sl, :], w_in_ref[...],
                        preferred_element_type=jnp.float32) + b_in_ref[...])

    def xh_chunk(c, slot):
        src = pl.ds(pl.multiple_of(c * rows_c, rows_c), rows_c)
        dst = pl.ds(pl.multiple_of(slot * rows_c, rows_c), rows_c)
        xh_ref[dst, :] = (
            jnp.dot(act_ref[src, :].astype(jnp.bfloat16), w_ih_ref[...],
                    preferred_element_type=jnp.float32) + b_ref[...])

    xh_chunk(jnp.int32(0), jnp.int32(0))

    def chunk_body(c, h):
        slot = c & 1
        base = slot * rows_c
        for t in range(chunk_t):
            r0 = pl.multiple_of(base + t * bh, bh)
            pre = (xh_ref[pl.ds(r0, bh), :]
                   + jnp.dot(h.astype(jnp.bfloat16), w_hh_ref[...],
                             preferred_element_type=jnp.float32))
            h = jnp.tanh(pre)
            hbuf_ref[t * bh:(t + 1) * bh, :] = h
        xh_chunk(jnp.minimum(c + 1, n_chunks - 1), 1 - slot)
        rows = pl.ds(pl.multiple_of(c * rows_c, rows_c), rows_c)
        act_ref[rows, :] = act_ref[rows, :] + hbuf_ref[...]
        return h

    h_last = lax.fori_loop(0, n_chunks, chunk_body,
                           jnp.zeros((bh, nhid), jnp.float32))
    hid_ref[0] = h_last

    @pl.when(l == nlayers - 1)
    def _():
        for c in range(n_chunks):
            sl = pl.ds(c * rows_c, rows_c)
            out_ref[sl, :] = (
                jnp.dot(act_ref[sl, :].astype(jnp.bfloat16), w_out_ref[...],
                        preferred_element_type=jnp.float32) + b_out_ref[...])


def kernel(x, w_in, b_in, w_ih, b_ih, w_hh, b_hh, w_out, b_out):
    B, T, ninp = x.shape
    nlayers, nhid, _ = w_ih.shape
    NC = 2
    bh = B // NC
    chunk_t = _chunk_t(T)
    n_chunks = T // chunk_t
    rows_c = chunk_t * bh

    emb = (x.transpose(1, 0, 2).reshape(T, NC, bh, ninp)
           .transpose(1, 0, 2, 3).reshape(NC, T * bh, ninp)
           .astype(jnp.bfloat16))

    w_in_b = w_in.astype(jnp.bfloat16)
    w_out_b = w_out.astype(jnp.bfloat16)
    w_ih_b = w_ih.astype(jnp.bfloat16)
    w_hh_b = w_hh.astype(jnp.bfloat16)
    b_sum = (b_ih + b_hh).astype(jnp.float32)

    fn = functools.partial(_fused_encoder_kernel, bh=bh, nhid=nhid,
                           chunk_t=chunk_t, n_chunks=n_chunks,
                           nlayers=nlayers)

    vmem_est = (T * bh * nhid * 4
                + 3 * rows_c * nhid * 4
                + T * bh * ninp * (2 + 4)
                + 4 * nhid * nhid * 2
                + 2 * (ninp * nhid * 2))
    vmem_limit = int(min(vmem_est + (16 << 20), 110 << 20))

    cost = pl.CostEstimate(
        flops=int(4 * T * B * ninp * nhid + 4 * nlayers * T * B * nhid * nhid),
        transcendentals=int(nlayers * T * B * nhid),
        bytes_accessed=int(2 * T * B * ninp * (2 + 4)
                           + nlayers * (4 * nhid * nhid + nhid * 4)
                           + nlayers * B * nhid * 4),
    )

    out2d, hidden = pl.pallas_call(
        fn,
        out_shape=(
            jax.ShapeDtypeStruct((NC, T * bh, ninp), jnp.float32),
            jax.ShapeDtypeStruct((nlayers, B, nhid), jnp.float32),
        ),
        grid=(NC, nlayers),
        in_specs=[
            pl.BlockSpec((None, T * bh, ninp), lambda p, l: (p, 0, 0)),
            pl.BlockSpec((ninp, nhid), lambda p, l: (0, 0)),
            pl.BlockSpec((1, nhid), lambda p, l: (0, 0)),
            pl.BlockSpec((None, nhid, nhid), lambda p, l: (l, 0, 0)),
            pl.BlockSpec((None, 1, nhid), lambda p, l: (l, 0, 0)),
            pl.BlockSpec((None, nhid, nhid), lambda p, l: (l, 0, 0)),
            pl.BlockSpec((nhid, ninp), lambda p, l: (0, 0)),
            pl.BlockSpec((1, ninp), lambda p, l: (0, 0)),
        ],
        out_specs=[
            pl.BlockSpec((None, T * bh, ninp), lambda p, l: (p, 0, 0)),
            pl.BlockSpec((1, bh, nhid), lambda p, l: (l, p, 0)),
        ],
        scratch_shapes=[
            pltpu.VMEM((T * bh, nhid), jnp.float32),
            pltpu.VMEM((2 * rows_c, nhid), jnp.float32),
            pltpu.VMEM((rows_c, nhid), jnp.float32),
        ],
        compiler_params=pltpu.CompilerParams(
            dimension_semantics=("parallel", "arbitrary"),
            vmem_limit_bytes=vmem_limit),
        cost_estimate=cost,
    )(emb, w_in_b, b_in, w_ih_b, b_sum, w_hh_b, w_out_b, b_out)

    out = (out2d.reshape(NC, T, bh, ninp).transpose(0, 2, 1, 3)
           .reshape(B, T, ninp))
    return out, hidden
